# baseline (speedup 1.0000x reference)
import functools

import jax
import jax.numpy as jnp
from jax import lax
from jax.experimental import pallas as pl
from jax.experimental.pallas import tpu as pltpu
from jax.experimental.pallas import tpu_sc as plsc

N, E, D, H, O, G = 10000, 320000, 128, 128, 8, 32

NC, NS, L = 2, 16, 16
NW = NC * NS
CHUNK = 128
KPB = 40
NBLK = 2
CHUNKS = KPB * NBLK
EPT = E // NW
EPT_PAD = CHUNKS * CHUNK
N_PAD = 10240
RPS = N_PAD // NS
ZR = 128


def _make_sc_agg(with_deg):
    out_type = [jax.ShapeDtypeStruct((NC, N_PAD, D), jnp.float32)]
    scratch = [
        pltpu.VMEM((KPB, CHUNK), jnp.int32),
        pltpu.VMEM((KPB, CHUNK), jnp.int32),
        pltpu.VMEM((CHUNK, D), jnp.float32),
        pltpu.VMEM((CHUNK, D), jnp.float32),
        pltpu.VMEM_SHARED((N_PAD, D), jnp.float32),
        pltpu.SemaphoreType.DMA,
        pltpu.SemaphoreType.DMA,
    ]
    if with_deg:
        out_type.append(jax.ShapeDtypeStruct((NC, N_PAD), jnp.float32))
        scratch += [
            pltpu.VMEM((CHUNK,), jnp.float32),
            pltpu.VMEM((RPS,), jnp.float32),
            pltpu.VMEM_SHARED((N_PAD,), jnp.float32),
            pltpu.SemaphoreType.DMA,
        ]

    def body(table_hbm, src_hbm, dst_hbm, zeros_hbm, ones_hbm, *refs):
        if with_deg:
            (agg_out, deg_out, src_v, dst_v, gbuf0, gbuf1, acc_sh, gsem,
             ssem, ones_v, z1buf, deg_sh, dsem) = refs
        else:
            agg_out, src_v, dst_v, gbuf0, gbuf1, acc_sh, gsem, ssem = refs
        cid = lax.axis_index("c")
        sid = lax.axis_index("s")
        wid = sid * NC + cid

        pltpu.sync_copy(zeros_hbm, gbuf0)

        def zero_body(i, carry):
            pltpu.sync_copy(gbuf0, acc_sh.at[pl.ds(sid * RPS + i * ZR, ZR)])
            return carry

        lax.fori_loop(0, RPS // ZR, zero_body, 0)
        if with_deg:
            pltpu.sync_copy(ones_hbm, ones_v)

            def z1_body(i, carry):
                z1buf[pl.ds(i * 16, 16)] = jnp.zeros((16,), jnp.float32)
                return carry

            lax.fori_loop(0, RPS // 16, z1_body, 0)
            pltpu.sync_copy(z1buf, deg_sh.at[pl.ds(sid * RPS, RPS)])
        plsc.subcore_barrier()

        for blk in range(NBLK):
            pltpu.sync_copy(src_hbm.at[wid, blk], src_v)
            pltpu.sync_copy(dst_hbm.at[wid, blk], dst_v)
            pltpu.async_copy(table_hbm.at[src_v.at[0]], gbuf0, gsem)

            def loop(i, carry):
                for gb, go, b in ((gbuf0, gbuf1, 0), (gbuf1, gbuf0, 1)):
                    c = i * 2 + b
                    pltpu.make_async_copy(table_hbm.at[src_v.at[c]], gb,
                                          gsem).wait()

                    @pl.when(c >= 1)
                    def _():
                        pltpu.make_async_copy(go,
                                              acc_sh.at[dst_v.at[c - 1]],
                                              ssem).wait()

                    @pl.when(c + 1 < KPB)
                    def _():
                        pltpu.async_copy(table_hbm.at[src_v.at[c + 1]], go,
                                         gsem)

                    pltpu.async_copy(gb, acc_sh.at[dst_v.at[c]], ssem,
                                     add=True)
                    if with_deg:
                        pltpu.async_copy(ones_v, deg_sh.at[dst_v.at[c]],
                                         dsem, add=True)
                return carry

            lax.fori_loop(0, KPB // 2, loop, 0)
            pltpu.make_async_copy(gbuf1, acc_sh.at[dst_v.at[KPB - 1]],
                                  ssem).wait()
            if with_deg:
                def drain(c, carry):
                    pltpu.make_async_copy(ones_v, deg_sh.at[dst_v.at[0]],
                                          dsem).wait()
                    return carry

                lax.fori_loop(0, KPB, drain, 0)
        plsc.subcore_barrier()

        pltpu.sync_copy(acc_sh.at[pl.ds(sid * RPS, RPS)],
                        agg_out.at[cid, pl.ds(sid * RPS, RPS)])
        if with_deg:
            pltpu.sync_copy(deg_sh.at[pl.ds(sid * RPS, RPS)],
                            deg_out.at[cid, pl.ds(sid * RPS, RPS)])

    return pl.kernel(
        body,
        out_type=out_type,
        mesh=plsc.VectorSubcoreMesh(core_axis_name="c", subcore_axis_name="s"),
        scratch_types=scratch,
    )


_sc_agg_deg = _make_sc_agg(True)
_sc_agg = _make_sc_agg(False)


_DN = (((1,), (1,)), ((), ()))


def _tc0_body(agg_ref, deg_ref, x_ref, w0l_ref, b0l_ref, w0r_ref, w1_ref,
              b1_ref, out_ref):
    agg = agg_ref[0, :N, :] + agg_ref[1, :N, :]
    deg = jnp.maximum(deg_ref[0, :N, :] + deg_ref[1, :N, :], 1.0)
    mean = agg / deg
    h0 = lax.dot_general(mean, w0l_ref[...], _DN,
                         preferred_element_type=jnp.float32)
    h0 = h0 + b0l_ref[...] + lax.dot_general(
        x_ref[...], w0r_ref[...], _DN, preferred_element_type=jnp.float32)
    h0 = jnp.maximum(h0, 0.0)
    h1 = lax.dot_general(h0, w1_ref[...], _DN,
                         preferred_element_type=jnp.float32) + b1_ref[...]
    out_ref[...] = jnp.maximum(h1, 0.0)


_tc0 = pl.pallas_call(
    _tc0_body,
    out_shape=jax.ShapeDtypeStruct((N, D), jnp.float32),
)


def _tc1_body(agg_ref, deg_ref, h1_ref, batch_ref, w2l_ref, b2l_ref, w2r_ref,
              fcw_ref, fcb_ref, out_ref):
    agg = agg_ref[0, :N, :] + agg_ref[1, :N, :]
    deg = jnp.maximum(deg_ref[0, :N, :] + deg_ref[1, :N, :], 1.0)
    mean = agg / deg
    h2 = lax.dot_general(mean, w2l_ref[...], _DN,
                         preferred_element_type=jnp.float32)
    h2 = h2 + b2l_ref[...] + lax.dot_general(
        h1_ref[...], w2r_ref[...], _DN, preferred_element_type=jnp.float32)
    h2 = jnp.maximum(h2, 0.0)
    gids = lax.broadcasted_iota(jnp.int32, (G, N), 0)
    onehot = (batch_ref[...] == gids).astype(jnp.float32)
    sums = lax.dot_general(onehot, h2, (((1,), (0,)), ((), ())),
                           preferred_element_type=jnp.float32)
    counts = jnp.sum(onehot, axis=1, keepdims=True)
    pooled = sums / jnp.maximum(counts, 1.0)
    out_ref[...] = lax.dot_general(pooled, fcw_ref[...], _DN,
                                   preferred_element_type=jnp.float32) \
        + fcb_ref[...]


_tc1 = pl.pallas_call(
    _tc1_body,
    out_shape=jax.ShapeDtypeStruct((G, O), jnp.float32),
)


def kernel(x, edge_index, batch, W0l, b0l, W0r, W1, b1, W2l, b2l, W2r,
           fcW, fcb):
    pad = EPT_PAD * NW - E
    src_p = jnp.concatenate([edge_index[0], jnp.zeros((pad,), jnp.int32)])
    dst_p = jnp.concatenate(
        [edge_index[1], jnp.full((pad,), N, jnp.int32)])
    src_p = src_p.reshape(NW, NBLK, KPB, CHUNK)
    dst_p = dst_p.reshape(NW, NBLK, KPB, CHUNK)
    zeros_blk = jnp.zeros((ZR, D), jnp.float32)
    ones_blk = jnp.ones((CHUNK,), jnp.float32)

    agg0, deg = _sc_agg_deg(x, src_p, dst_p, zeros_blk, ones_blk)
    deg = deg.reshape(NC, N_PAD, 1)
    h1 = _tc0(agg0, deg, x, W0l, b0l.reshape(1, H), W0r, W1,
              b1.reshape(1, H))
    (agg2,) = _sc_agg(h1, src_p, dst_p, zeros_blk, ones_blk)
    out = _tc1(agg2, deg, h1, batch.reshape(1, N).astype(jnp.int32),
               W2l, b2l.reshape(1, H), W2r, fcW, fcb.reshape(1, O))
    return out


# speedup vs baseline: 1.0874x; 1.0874x over previous
import functools

import jax
import jax.numpy as jnp
from jax import lax
from jax.experimental import pallas as pl
from jax.experimental.pallas import tpu as pltpu
from jax.experimental.pallas import tpu_sc as plsc

N, E, D, H, O, G = 10000, 320000, 128, 128, 8, 32

NC, NS, L = 2, 16, 16
NW = NC * NS
CHUNK = 128
KPB = 40
NBLK = 2
CHUNKS = KPB * NBLK
EPT = E // NW
EPT_PAD = CHUNKS * CHUNK
N_PAD = 10240
RPS = N_PAD // NS
ZR = 128


def _make_sc_agg(with_deg):
    out_type = [jax.ShapeDtypeStruct((NC, N_PAD, D), jnp.float32)]
    scratch = [
        pltpu.VMEM((KPB, CHUNK), jnp.int32),
        pltpu.VMEM((KPB, CHUNK), jnp.int32),
        pltpu.VMEM((CHUNK, D), jnp.float32),
        pltpu.VMEM((CHUNK, D), jnp.float32),
        pltpu.VMEM_SHARED((N_PAD, D), jnp.float32),
        pltpu.SemaphoreType.DMA,
        pltpu.SemaphoreType.DMA,
    ]
    if with_deg:
        out_type.append(jax.ShapeDtypeStruct((NC, N_PAD), jnp.float32))
        scratch += [
            pltpu.VMEM((CHUNK,), jnp.float32),
            pltpu.VMEM((RPS,), jnp.float32),
            pltpu.VMEM_SHARED((N_PAD,), jnp.float32),
            pltpu.SemaphoreType.DMA,
        ]

    def body(table_hbm, src_hbm, dst_hbm, zeros_hbm, ones_hbm, *refs):
        if with_deg:
            (agg_out, deg_out, src_v, dst_v, gbuf0, gbuf1, acc_sh, gsem,
             ssem, ones_v, z1buf, deg_sh, dsem) = refs
        else:
            agg_out, src_v, dst_v, gbuf0, gbuf1, acc_sh, gsem, ssem = refs
        cid = lax.axis_index("c")
        sid = lax.axis_index("s")
        wid = sid * NC + cid

        pltpu.sync_copy(zeros_hbm, gbuf0)

        def zero_body(i, carry):
            pltpu.sync_copy(gbuf0, acc_sh.at[pl.ds(sid * RPS + i * ZR, ZR)])
            return carry

        lax.fori_loop(0, RPS // ZR, zero_body, 0)
        if with_deg:
            pltpu.sync_copy(ones_hbm, ones_v)

            def z1_body(i, carry):
                z1buf[pl.ds(i * 16, 16)] = jnp.zeros((16,), jnp.float32)
                return carry

            lax.fori_loop(0, RPS // 16, z1_body, 0)
            pltpu.sync_copy(z1buf, deg_sh.at[pl.ds(sid * RPS, RPS)])
        plsc.subcore_barrier()

        for blk in range(NBLK):
            pltpu.sync_copy(src_hbm.at[wid, blk], src_v)
            pltpu.sync_copy(dst_hbm.at[wid, blk], dst_v)
            pltpu.async_copy(table_hbm.at[src_v.at[0]], gbuf0, gsem)

            def loop(i, carry):
                for gb, go, b in ((gbuf0, gbuf1, 0), (gbuf1, gbuf0, 1)):
                    c = i * 2 + b
                    pltpu.make_async_copy(table_hbm.at[src_v.at[c]], gb,
                                          gsem).wait()

                    @pl.when(c >= 1)
                    def _():
                        pltpu.make_async_copy(go,
                                              acc_sh.at[dst_v.at[c - 1]],
                                              ssem).wait()

                    @pl.when(c + 1 < KPB)
                    def _():
                        pltpu.async_copy(table_hbm.at[src_v.at[c + 1]], go,
                                         gsem)

                    pltpu.async_copy(gb, acc_sh.at[dst_v.at[c]], ssem,
                                     add=True)
                    if with_deg:
                        pltpu.async_copy(ones_v, deg_sh.at[dst_v.at[c]],
                                         dsem, add=True)
                return carry

            lax.fori_loop(0, KPB // 2, loop, 0)
            pltpu.make_async_copy(gbuf1, acc_sh.at[dst_v.at[KPB - 1]],
                                  ssem).wait()
            if with_deg:
                def drain(c, carry):
                    pltpu.make_async_copy(ones_v, deg_sh.at[dst_v.at[0]],
                                          dsem).wait()
                    return carry

                lax.fori_loop(0, KPB, drain, 0)
        plsc.subcore_barrier()

        pltpu.sync_copy(acc_sh.at[pl.ds(sid * RPS, RPS)],
                        agg_out.at[cid, pl.ds(sid * RPS, RPS)])
        if with_deg:
            pltpu.sync_copy(deg_sh.at[pl.ds(sid * RPS, RPS)],
                            deg_out.at[cid, pl.ds(sid * RPS, RPS)])

    return pl.kernel(
        body,
        out_type=out_type,
        mesh=plsc.VectorSubcoreMesh(core_axis_name="c", subcore_axis_name="s"),
        scratch_types=scratch,
    )


_sc_agg_deg = _make_sc_agg(True)
_sc_agg = _make_sc_agg(False)


_DN = (((1,), (1,)), ((), ()))


def _tc0_body(agg_ref, deg_ref, x_ref, w0l_ref, b0l_ref, w0r_ref, w1_ref,
              b1_ref, out_ref):
    agg = agg_ref[0, :N, :] + agg_ref[1, :N, :]
    deg = jnp.maximum(deg_ref[0, :N, :] + deg_ref[1, :N, :], 1.0)
    mean = agg / deg
    h0 = lax.dot_general(mean, w0l_ref[...], _DN,
                         preferred_element_type=jnp.float32)
    h0 = h0 + b0l_ref[...] + lax.dot_general(
        x_ref[...], w0r_ref[...], _DN, preferred_element_type=jnp.float32)
    h0 = jnp.maximum(h0, 0.0)
    h1 = lax.dot_general(h0, w1_ref[...], _DN,
                         preferred_element_type=jnp.float32) + b1_ref[...]
    out_ref[...] = jnp.maximum(h1, 0.0)


_tc0 = pl.pallas_call(
    _tc0_body,
    out_shape=jax.ShapeDtypeStruct((N, D), jnp.float32),
)


def _tc1_body(agg_ref, deg_ref, h1_ref, batch_ref, w2l_ref, b2l_ref, w2r_ref,
              fcw_ref, fcb_ref, out_ref):
    agg = agg_ref[0, :N, :] + agg_ref[1, :N, :]
    deg = jnp.maximum(deg_ref[0, :N, :] + deg_ref[1, :N, :], 1.0)
    mean = agg / deg
    h2 = lax.dot_general(mean, w2l_ref[...], _DN,
                         preferred_element_type=jnp.float32)
    h2 = h2 + b2l_ref[...] + lax.dot_general(
        h1_ref[...], w2r_ref[...], _DN, preferred_element_type=jnp.float32)
    h2 = jnp.maximum(h2, 0.0)
    gids = lax.broadcasted_iota(jnp.int32, (G, N), 0)
    onehot = (batch_ref[...] == gids).astype(jnp.float32)
    sums = lax.dot_general(onehot, h2, (((1,), (0,)), ((), ())),
                           preferred_element_type=jnp.float32)
    counts = jnp.sum(onehot, axis=1, keepdims=True)
    pooled = sums / jnp.maximum(counts, 1.0)
    out_ref[...] = lax.dot_general(pooled, fcw_ref[...], _DN,
                                   preferred_element_type=jnp.float32) \
        + fcb_ref[...]


_tc1 = pl.pallas_call(
    _tc1_body,
    out_shape=jax.ShapeDtypeStruct((G, O), jnp.float32),
)


def kernel(x, edge_index, batch, W0l, b0l, W0r, W1, b1, W2l, b2l, W2r,
           fcW, fcb):
    pad = EPT_PAD - EPT
    pad_dst = jnp.broadcast_to(N + (jnp.arange(pad) % (N_PAD - N)),
                               (NW, pad)).astype(jnp.int32)
    pad_src = jnp.zeros((NW, pad), jnp.int32)
    src_p = jnp.concatenate([edge_index[0].reshape(NW, EPT), pad_src], 1)
    dst_p = jnp.concatenate([edge_index[1].reshape(NW, EPT), pad_dst], 1)
    src_p = src_p.reshape(NW, NBLK, KPB, CHUNK)
    dst_p = dst_p.reshape(NW, NBLK, KPB, CHUNK)
    zeros_blk = jnp.zeros((ZR, D), jnp.float32)
    ones_blk = jnp.ones((CHUNK,), jnp.float32)

    agg0, deg = _sc_agg_deg(x, src_p, dst_p, zeros_blk, ones_blk)
    deg = deg.reshape(NC, N_PAD, 1)
    h1 = _tc0(agg0, deg, x, W0l, b0l.reshape(1, H), W0r, W1,
              b1.reshape(1, H))
    (agg2,) = _sc_agg(h1, src_p, dst_p, zeros_blk, ones_blk)
    out = _tc1(agg2, deg, h1, batch.reshape(1, N).astype(jnp.int32),
               W2l, b2l.reshape(1, H), W2r, fcW, fcb.reshape(1, O))
    return out


# speedup vs baseline: 1.4550x; 1.3380x over previous
import functools

import jax
import jax.numpy as jnp
from jax import lax
from jax.experimental import pallas as pl
from jax.experimental.pallas import tpu as pltpu
from jax.experimental.pallas import tpu_sc as plsc

N, E, D, H, O, G = 10000, 320000, 128, 128, 8, 32

NC, NS, L = 2, 16, 16
NW = NC * NS
CHUNK = 128
EPT = E // NW
CHUNKS = (EPT + CHUNK - 1) // CHUNK
EPT_PAD = CHUNKS * CHUNK
N_PAD = 10240
RPS = N_PAD // NS
ZR = 64


def _make_sc_agg(with_deg):
    out_type = [jax.ShapeDtypeStruct((NC, N_PAD, D), jnp.float32)]
    scratch = [
        pltpu.VMEM((CHUNKS, CHUNK), jnp.int32),
        pltpu.VMEM((CHUNKS, CHUNK), jnp.int32),
        pltpu.VMEM((CHUNK, D), jnp.float32),
        pltpu.VMEM((ZR, D), jnp.float32),
        pltpu.VMEM_SHARED((N_PAD, D), jnp.float32),
        pltpu.SemaphoreType.DMA,
        pltpu.SemaphoreType.DMA,
    ]
    if with_deg:
        out_type.append(jax.ShapeDtypeStruct((NC, N_PAD), jnp.float32))
        scratch += [
            pltpu.VMEM((CHUNK,), jnp.float32),
            pltpu.VMEM((RPS,), jnp.float32),
            pltpu.VMEM_SHARED((N_PAD,), jnp.float32),
            pltpu.SemaphoreType.DMA,
        ]

    def body(table_hbm, src_hbm, dst_hbm, zeros_hbm, ones_hbm, *refs):
        if with_deg:
            (agg_out, deg_out, src_v, dst_v, gbuf, zbuf, acc_sh, gsem, ssem,
             ones_v, z1buf, deg_sh, dsem) = refs
        else:
            agg_out, src_v, dst_v, gbuf, zbuf, acc_sh, gsem, ssem = refs
        cid = lax.axis_index("c")
        sid = lax.axis_index("s")
        wid = sid * NC + cid

        pltpu.sync_copy(src_hbm.at[wid], src_v)
        pltpu.sync_copy(dst_hbm.at[wid], dst_v)

        pltpu.sync_copy(zeros_hbm, zbuf)

        def zero_body(i, carry):
            pltpu.sync_copy(zbuf, acc_sh.at[pl.ds(sid * RPS + i * ZR, ZR)])
            return carry

        lax.fori_loop(0, RPS // ZR, zero_body, 0)
        if with_deg:
            pltpu.sync_copy(ones_hbm, ones_v)

            def z1_body(i, carry):
                z1buf[pl.ds(i * 16, 16)] = jnp.zeros((16,), jnp.float32)
                return carry

            lax.fori_loop(0, RPS // 16, z1_body, 0)
            pltpu.sync_copy(z1buf, deg_sh.at[pl.ds(sid * RPS, RPS)])
        plsc.subcore_barrier()

        def loop(c, carry):
            pltpu.async_copy(table_hbm.at[src_v.at[c]], gbuf, gsem).wait()
            scat = pltpu.async_copy(gbuf, acc_sh.at[dst_v.at[c]], ssem,
                                    add=True)
            if with_deg:
                pltpu.async_copy(ones_v, deg_sh.at[dst_v.at[c]], dsem,
                                 add=True).wait()
            scat.wait()
            return carry

        lax.fori_loop(0, CHUNKS, loop, 0)
        plsc.subcore_barrier()

        pltpu.sync_copy(acc_sh.at[pl.ds(sid * RPS, RPS)],
                        agg_out.at[cid, pl.ds(sid * RPS, RPS)])
        if with_deg:
            pltpu.sync_copy(deg_sh.at[pl.ds(sid * RPS, RPS)],
                            deg_out.at[cid, pl.ds(sid * RPS, RPS)])

    return pl.kernel(
        body,
        out_type=out_type,
        mesh=plsc.VectorSubcoreMesh(core_axis_name="c", subcore_axis_name="s"),
        scratch_types=scratch,
    )


_sc_agg_deg = _make_sc_agg(True)
_sc_agg = _make_sc_agg(False)


_DN = (((1,), (1,)), ((), ()))


def _tc0_body(agg_ref, deg_ref, x_ref, w0l_ref, b0l_ref, w0r_ref, w1_ref,
              b1_ref, out_ref):
    agg = agg_ref[0, :N, :] + agg_ref[1, :N, :]
    deg = jnp.maximum(deg_ref[0, :N, :] + deg_ref[1, :N, :], 1.0)
    mean = agg / deg
    h0 = lax.dot_general(mean, w0l_ref[...], _DN,
                         preferred_element_type=jnp.float32)
    h0 = h0 + b0l_ref[...] + lax.dot_general(
        x_ref[...], w0r_ref[...], _DN, preferred_element_type=jnp.float32)
    h0 = jnp.maximum(h0, 0.0)
    h1 = lax.dot_general(h0, w1_ref[...], _DN,
                         preferred_element_type=jnp.float32) + b1_ref[...]
    out_ref[...] = jnp.maximum(h1, 0.0)


_tc0 = pl.pallas_call(
    _tc0_body,
    out_shape=jax.ShapeDtypeStruct((N, D), jnp.float32),
)


def _tc1_body(agg_ref, deg_ref, h1_ref, batch_ref, w2l_ref, b2l_ref, w2r_ref,
              fcw_ref, fcb_ref, out_ref):
    agg = agg_ref[0, :N, :] + agg_ref[1, :N, :]
    deg = jnp.maximum(deg_ref[0, :N, :] + deg_ref[1, :N, :], 1.0)
    mean = agg / deg
    h2 = lax.dot_general(mean, w2l_ref[...], _DN,
                         preferred_element_type=jnp.float32)
    h2 = h2 + b2l_ref[...] + lax.dot_general(
        h1_ref[...], w2r_ref[...], _DN, preferred_element_type=jnp.float32)
    h2 = jnp.maximum(h2, 0.0)
    gids = lax.broadcasted_iota(jnp.int32, (G, N), 0)
    onehot = (batch_ref[...] == gids).astype(jnp.float32)
    sums = lax.dot_general(onehot, h2, (((1,), (0,)), ((), ())),
                           preferred_element_type=jnp.float32)
    counts = jnp.sum(onehot, axis=1, keepdims=True)
    pooled = sums / jnp.maximum(counts, 1.0)
    out_ref[...] = lax.dot_general(pooled, fcw_ref[...], _DN,
                                   preferred_element_type=jnp.float32) \
        + fcb_ref[...]


_tc1 = pl.pallas_call(
    _tc1_body,
    out_shape=jax.ShapeDtypeStruct((G, O), jnp.float32),
)


def kernel(x, edge_index, batch, W0l, b0l, W0r, W1, b1, W2l, b2l, W2r,
           fcW, fcb):
    pad = EPT_PAD - EPT
    pad_dst = jnp.broadcast_to(N + (jnp.arange(pad) % (N_PAD - N)),
                               (NW, pad)).astype(jnp.int32)
    pad_src = jnp.zeros((NW, pad), jnp.int32)
    src_p = jnp.concatenate([edge_index[0].reshape(NW, EPT), pad_src], 1)
    dst_p = jnp.concatenate([edge_index[1].reshape(NW, EPT), pad_dst], 1)
    src_p = src_p.reshape(NW, CHUNKS, CHUNK)
    dst_p = dst_p.reshape(NW, CHUNKS, CHUNK)
    zeros_blk = jnp.zeros((ZR, D), jnp.float32)
    ones_blk = jnp.ones((CHUNK,), jnp.float32)

    agg0, deg = _sc_agg_deg(x, src_p, dst_p, zeros_blk, ones_blk)
    deg = deg.reshape(NC, N_PAD, 1)
    h1 = _tc0(agg0, deg, x, W0l, b0l.reshape(1, H), W0r, W1,
              b1.reshape(1, H))
    (agg2,) = _sc_agg(h1, src_p, dst_p, zeros_blk, ones_blk)
    out = _tc1(agg2, deg, h1, batch.reshape(1, N).astype(jnp.int32),
               W2l, b2l.reshape(1, H), W2r, fcW, fcb.reshape(1, O))
    return out
